# Initial kernel scaffold; baseline (speedup 1.0000x reference)
import jax
import jax.numpy as jnp
from jax.experimental import pallas as pl


def kernel(id_emb, llm_emb, W1, b1, W2, b2, Wg1, bg1, Wg2, bg2):
    raise NotImplementedError("write your pallas kernel here")


# speedup vs baseline: 2.4974x; 2.4974x over previous
import functools

import jax
import jax.numpy as jnp
from jax.experimental import pallas as pl

_ID_DIM = 32
_LLM_DIM = 768
_OUT_DIM = 32
_E = 8
_H = 2 * _OUT_DIM
_B = 16384
_BT = 2048


def _fused_body(id_ref, llm_ref, w1a_ref, w1b_ref, b1_ref, w2_ref, b2_ref,
                wg1a_ref, wg1b_ref, bg1_ref, wg2_ref, bg2_ref, exp_ref,
                out_ref):
    idb = id_ref[...]
    llm = llm_ref[...]
    f32 = jnp.float32

    gh = jnp.maximum(
        jnp.dot(idb, wg1a_ref[...], preferred_element_type=f32)
        + jnp.dot(llm, wg1b_ref[...], preferred_element_type=f32)
        + bg1_ref[...], 0.0)
    logits = jnp.dot(gh, wg2_ref[...], preferred_element_type=f32) + bg2_ref[...]

    lane = jax.lax.broadcasted_iota(jnp.int32, logits.shape, 1)
    m1 = jnp.max(logits, axis=-1, keepdims=True)
    i1 = jnp.min(jnp.where(logits == m1, lane, _E), axis=-1, keepdims=True)
    oh1 = lane == i1
    masked = jnp.where(oh1, -jnp.inf, logits)
    m2 = jnp.max(masked, axis=-1, keepdims=True)
    i2 = jnp.min(jnp.where(masked == m2, lane, _E), axis=-1, keepdims=True)
    oh2 = lane == i2
    wtop = 1.0 / (1.0 + jnp.exp(m2 - m1))
    wvec = jnp.where(oh1, wtop, 0.0) + jnp.where(oh2, 1.0 - wtop, 0.0)

    h = jnp.maximum(
        jnp.dot(idb, w1a_ref[...], preferred_element_type=f32)
        + jnp.dot(llm, w1b_ref[...], preferred_element_type=f32)
        + b1_ref[...], 0.0)

    wexp = jnp.dot(wvec, exp_ref[...], preferred_element_type=f32)
    out = jnp.dot(h * wexp, w2_ref[...], preferred_element_type=f32)
    out_ref[...] = out + jnp.dot(wvec, b2_ref[...], preferred_element_type=f32)


@functools.partial(jax.jit, static_argnames=())
def kernel(id_emb, llm_emb, W1, b1, W2, b2, Wg1, bg1, Wg2, bg2):
    EH = _E * _H
    w1_flat = jnp.transpose(W1, (1, 0, 2)).reshape(_ID_DIM + _LLM_DIM, EH)
    w1a, w1b = w1_flat[:_ID_DIM], w1_flat[_ID_DIM:]
    b1_flat = b1.reshape(1, EH)
    w2_flat = W2.reshape(EH, _OUT_DIM)
    wg1a, wg1b = Wg1[:_ID_DIM], Wg1[_ID_DIM:]
    exp_mat = jnp.repeat(jnp.eye(_E, dtype=jnp.float32), _H, axis=1)

    full = lambda shape: pl.BlockSpec(shape, lambda i: (0, 0))
    grid = (_B // _BT,)
    return pl.pallas_call(
        _fused_body,
        grid=grid,
        in_specs=[
            pl.BlockSpec((_BT, _ID_DIM), lambda i: (i, 0)),
            pl.BlockSpec((_BT, _LLM_DIM), lambda i: (i, 0)),
            full((_ID_DIM, EH)),
            full((_LLM_DIM, EH)),
            full((1, EH)),
            full((EH, _OUT_DIM)),
            full((_E, _OUT_DIM)),
            full((_ID_DIM, 2 * _E)),
            full((_LLM_DIM, 2 * _E)),
            full((1, 2 * _E)),
            full((2 * _E, _E)),
            full((1, _E)),
            full((_E, EH)),
        ],
        out_specs=pl.BlockSpec((_BT, _OUT_DIM), lambda i: (i, 0)),
        out_shape=jax.ShapeDtypeStruct((_B, _OUT_DIM), jnp.float32),
    )(id_emb, llm_emb, w1a, w1b, b1_flat, w2_flat, b2,
      wg1a, wg1b, bg1.reshape(1, 2 * _E), Wg2, bg2.reshape(1, _E), exp_mat)


# speedup vs baseline: 2.5006x; 1.0013x over previous
import functools

import jax
import jax.numpy as jnp
from jax.experimental import pallas as pl

_ID_DIM = 32
_LLM_DIM = 768
_OUT_DIM = 32
_E = 8
_H = 2 * _OUT_DIM
_B = 16384
_BT = 2048


def _fused_body(id_ref, llm_ref, w1a_ref, w1b_ref, b1_ref, w2_ref, b2_ref,
                wg1a_ref, wg1b_ref, bg1_ref, wg2_ref, bg2_ref, exp_ref,
                out_ref):
    idb = id_ref[...]
    llm = llm_ref[...]
    f32 = jnp.float32

    gh = jnp.maximum(
        jnp.dot(idb, wg1a_ref[...], preferred_element_type=f32)
        + jnp.dot(llm, wg1b_ref[...], preferred_element_type=f32)
        + bg1_ref[...], 0.0)
    logits = jnp.dot(gh, wg2_ref[...], preferred_element_type=f32) + bg2_ref[...]

    lane = jax.lax.broadcasted_iota(jnp.int32, logits.shape, 1)
    m1 = jnp.max(logits, axis=-1, keepdims=True)
    i1 = jnp.min(jnp.where(logits == m1, lane, _E), axis=-1, keepdims=True)
    oh1 = lane == i1
    masked = jnp.where(oh1, -jnp.inf, logits)
    m2 = jnp.max(masked, axis=-1, keepdims=True)
    i2 = jnp.min(jnp.where(masked == m2, lane, _E), axis=-1, keepdims=True)
    oh2 = lane == i2
    wtop = 1.0 / (1.0 + jnp.exp(m2 - m1))
    wvec = jnp.where(oh1, wtop, 0.0) + jnp.where(oh2, 1.0 - wtop, 0.0)

    bf16 = jnp.bfloat16
    h = jnp.maximum(
        jnp.dot(idb.astype(bf16), w1a_ref[...], preferred_element_type=f32)
        + jnp.dot(llm.astype(bf16), w1b_ref[...], preferred_element_type=f32)
        + b1_ref[...], 0.0)

    wexp = jnp.dot(wvec, exp_ref[...], preferred_element_type=f32)
    out = jnp.dot((h * wexp).astype(bf16), w2_ref[...],
                  preferred_element_type=f32)
    out_ref[...] = out + jnp.dot(wvec, b2_ref[...], preferred_element_type=f32)


@functools.partial(jax.jit, static_argnames=())
def kernel(id_emb, llm_emb, W1, b1, W2, b2, Wg1, bg1, Wg2, bg2):
    EH = _E * _H
    w1_flat = jnp.transpose(W1, (1, 0, 2)).reshape(_ID_DIM + _LLM_DIM, EH)
    w1a = w1_flat[:_ID_DIM].astype(jnp.bfloat16)
    w1b = w1_flat[_ID_DIM:].astype(jnp.bfloat16)
    b1_flat = b1.reshape(1, EH)
    w2_flat = W2.reshape(EH, _OUT_DIM).astype(jnp.bfloat16)
    wg1a, wg1b = Wg1[:_ID_DIM], Wg1[_ID_DIM:]
    exp_mat = jnp.repeat(jnp.eye(_E, dtype=jnp.float32), _H, axis=1)

    full = lambda shape: pl.BlockSpec(shape, lambda i: (0, 0))
    grid = (_B // _BT,)
    return pl.pallas_call(
        _fused_body,
        grid=grid,
        in_specs=[
            pl.BlockSpec((_BT, _ID_DIM), lambda i: (i, 0)),
            pl.BlockSpec((_BT, _LLM_DIM), lambda i: (i, 0)),
            full((_ID_DIM, EH)),
            full((_LLM_DIM, EH)),
            full((1, EH)),
            full((EH, _OUT_DIM)),
            full((_E, _OUT_DIM)),
            full((_ID_DIM, 2 * _E)),
            full((_LLM_DIM, 2 * _E)),
            full((1, 2 * _E)),
            full((2 * _E, _E)),
            full((1, _E)),
            full((_E, EH)),
        ],
        out_specs=pl.BlockSpec((_BT, _OUT_DIM), lambda i: (i, 0)),
        out_shape=jax.ShapeDtypeStruct((_B, _OUT_DIM), jnp.float32),
    )(id_emb, llm_emb, w1a, w1b, b1_flat, w2_flat, b2,
      wg1a, wg1b, bg1.reshape(1, 2 * _E), Wg2, bg2.reshape(1, _E), exp_mat)


# speedup vs baseline: 2.7278x; 1.0909x over previous
import functools

import jax
import jax.numpy as jnp
from jax.experimental import pallas as pl

_ID_DIM = 32
_LLM_DIM = 768
_OUT_DIM = 32
_E = 8
_H = 2 * _OUT_DIM
_EH = _E * _H
_GH = 2 * _E
_B = 16384
_BT = 2048


def _fused_body(id_ref, llm_ref, wa_ref, wb_ref, bias_ref, w2_ref, b2_ref,
                wg2t_ref, bg2_ref, exp_ref, out_ref):
    idb = id_ref[...]
    llm = llm_ref[...]
    f32 = jnp.float32

    hall = jnp.maximum(
        jnp.dot(idb, wa_ref[...], preferred_element_type=f32)
        + jnp.dot(llm, wb_ref[...], preferred_element_type=f32)
        + bias_ref[...], 0.0)
    h = hall[:, :_EH]
    ght = hall[:, _EH:].T

    logits = jnp.dot(wg2t_ref[...], ght, preferred_element_type=f32) + bg2_ref[...]

    sub = jax.lax.broadcasted_iota(jnp.int32, logits.shape, 0)
    m1 = jnp.max(logits, axis=0, keepdims=True)
    i1 = jnp.min(jnp.where(logits == m1, sub, _E), axis=0, keepdims=True)
    oh1 = sub == i1
    masked = jnp.where(oh1, -jnp.inf, logits)
    m2 = jnp.max(masked, axis=0, keepdims=True)
    i2 = jnp.min(jnp.where(masked == m2, sub, _E), axis=0, keepdims=True)
    oh2 = sub == i2
    wtop = 1.0 / (1.0 + jnp.exp(m2 - m1))
    wvec = (jnp.where(oh1, wtop, 0.0) + jnp.where(oh2, 1.0 - wtop, 0.0)).T

    wexp = jnp.dot(wvec, exp_ref[...], preferred_element_type=f32)
    out = jnp.dot(h * wexp, w2_ref[...], preferred_element_type=f32)
    out_ref[...] = out + jnp.dot(wvec, b2_ref[...], preferred_element_type=f32)


@functools.partial(jax.jit, static_argnames=())
def kernel(id_emb, llm_emb, W1, b1, W2, b2, Wg1, bg1, Wg2, bg2):
    D = _ID_DIM + _LLM_DIM
    w1_flat = jnp.transpose(W1, (1, 0, 2)).reshape(D, _EH)
    w_all = jnp.concatenate([w1_flat, Wg1], axis=1)
    wa, wb = w_all[:_ID_DIM], w_all[_ID_DIM:]
    bias = jnp.concatenate([b1.reshape(_EH), bg1]).reshape(1, _EH + _GH)
    w2_flat = W2.reshape(_EH, _OUT_DIM)
    exp_mat = jnp.repeat(jnp.eye(_E, dtype=jnp.float32), _H, axis=1)

    full = lambda shape: pl.BlockSpec(shape, lambda i: (0, 0))
    grid = (_B // _BT,)
    return pl.pallas_call(
        _fused_body,
        grid=grid,
        in_specs=[
            pl.BlockSpec((_BT, _ID_DIM), lambda i: (i, 0)),
            pl.BlockSpec((_BT, _LLM_DIM), lambda i: (i, 0)),
            full((_ID_DIM, _EH + _GH)),
            full((_LLM_DIM, _EH + _GH)),
            full((1, _EH + _GH)),
            full((_EH, _OUT_DIM)),
            full((_E, _OUT_DIM)),
            full((_E, _GH)),
            full((_E, 1)),
            full((_E, _EH)),
        ],
        out_specs=pl.BlockSpec((_BT, _OUT_DIM), lambda i: (i, 0)),
        out_shape=jax.ShapeDtypeStruct((_B, _OUT_DIM), jnp.float32),
    )(id_emb, llm_emb, wa, wb, bias, w2_flat, b2,
      Wg2.T, bg2.reshape(_E, 1), exp_mat)


# speedup vs baseline: 2.7317x; 1.0014x over previous
import functools

import jax
import jax.numpy as jnp
from jax.experimental import pallas as pl
from jax.experimental.pallas import tpu as pltpu

_ID_DIM = 32
_LLM_DIM = 768
_OUT_DIM = 32
_E = 8
_H = 2 * _OUT_DIM
_EH = _E * _H
_GH = 2 * _E
_B = 16384
_BT = 2048


def _fused_body(id_ref, llm_ref, wa_ref, wb_ref, bias_ref, w2_ref, b2_ref,
                wg2t_ref, bg2_ref, exp_ref, out_ref):
    idb = id_ref[...]
    llm = llm_ref[...]
    f32 = jnp.float32

    hall = jnp.maximum(
        jnp.dot(idb, wa_ref[...], preferred_element_type=f32)
        + jnp.dot(llm, wb_ref[...], preferred_element_type=f32)
        + bias_ref[...], 0.0)
    h = hall[:, :_EH]
    ght = hall[:, _EH:].T

    logits = jnp.dot(wg2t_ref[...], ght, preferred_element_type=f32) + bg2_ref[...]

    sub = jax.lax.broadcasted_iota(jnp.int32, logits.shape, 0)
    m1 = jnp.max(logits, axis=0, keepdims=True)
    i1 = jnp.min(jnp.where(logits == m1, sub, _E), axis=0, keepdims=True)
    oh1 = sub == i1
    masked = jnp.where(oh1, -jnp.inf, logits)
    m2 = jnp.max(masked, axis=0, keepdims=True)
    i2 = jnp.min(jnp.where(masked == m2, sub, _E), axis=0, keepdims=True)
    oh2 = sub == i2
    wtop = 1.0 / (1.0 + jnp.exp(m2 - m1))
    wvec = (jnp.where(oh1, wtop, 0.0) + jnp.where(oh2, 1.0 - wtop, 0.0)).T

    wexp = jnp.dot(wvec, exp_ref[...], preferred_element_type=f32)
    out = jnp.dot(h * wexp, w2_ref[...], preferred_element_type=f32)
    out_ref[...] = out + jnp.dot(wvec, b2_ref[...], preferred_element_type=f32)


@functools.partial(jax.jit, static_argnames=())
def kernel(id_emb, llm_emb, W1, b1, W2, b2, Wg1, bg1, Wg2, bg2):
    D = _ID_DIM + _LLM_DIM
    w1_flat = jnp.transpose(W1, (1, 0, 2)).reshape(D, _EH)
    w_all = jnp.concatenate([w1_flat, Wg1], axis=1)
    wa, wb = w_all[:_ID_DIM], w_all[_ID_DIM:]
    bias = jnp.concatenate([b1.reshape(_EH), bg1]).reshape(1, _EH + _GH)
    w2_flat = W2.reshape(_EH, _OUT_DIM)
    exp_mat = jnp.repeat(jnp.eye(_E, dtype=jnp.float32), _H, axis=1)

    full = lambda shape: pl.BlockSpec(shape, lambda i: (0, 0))
    grid = (_B // _BT,)
    return pl.pallas_call(
        _fused_body,
        grid=grid,
        in_specs=[
            pl.BlockSpec((_BT, _ID_DIM), lambda i: (i, 0)),
            pl.BlockSpec((_BT, _LLM_DIM), lambda i: (i, 0)),
            full((_ID_DIM, _EH + _GH)),
            full((_LLM_DIM, _EH + _GH)),
            full((1, _EH + _GH)),
            full((_EH, _OUT_DIM)),
            full((_E, _OUT_DIM)),
            full((_E, _GH)),
            full((_E, 1)),
            full((_E, _EH)),
        ],
        out_specs=pl.BlockSpec((_BT, _OUT_DIM), lambda i: (i, 0)),
        out_shape=jax.ShapeDtypeStruct((_B, _OUT_DIM), jnp.float32),
        compiler_params=pltpu.CompilerParams(
            dimension_semantics=("parallel",)),
    )(id_emb, llm_emb, wa, wb, bias, w2_flat, b2,
      Wg2.T, bg2.reshape(_E, 1), exp_mat)
